# Initial kernel scaffold; baseline (speedup 1.0000x reference)
import jax
import jax.numpy as jnp
from jax.experimental import pallas as pl


def kernel(mol_x, mol_edge_index, mol_batch, clique_x, clique_edge_index, clique_batch, data_pre, params):
    raise NotImplementedError("write your pallas kernel here")


# speedup vs baseline: 2.4731x; 2.4731x over previous
import functools

import jax
import jax.numpy as jnp
from jax import lax
from jax.experimental import pallas as pl
from jax.experimental.pallas import tpu as pltpu
from jax.experimental.pallas import tpu_sc as plsc

N_NODE = 10000
NP = 10240
E = 160000
NW = 32
EPT = 5008
E_PAD = NW * EPT
GRP = EPT // 16
B = 512
CH = 1024
NCHUNK = NP // CH
CH_SHIFT = 10
DUMMY = NP - 8
NEG = -1e30

_mesh = plsc.VectorSubcoreMesh(core_axis_name="c", subcore_axis_name="s")


def _wid():
    return lax.axis_index("s") * 2 + lax.axis_index("c")


def _mm_body(x_ref, w_ref, b_ref, o_ref, *, act):
    y = lax.dot_general(x_ref[...], w_ref[...], (((1,), (1,)), ((), ())),
                        preferred_element_type=jnp.float32)
    y = y + b_ref[...]
    if act == "relu":
        y = jnp.maximum(y, 0.0)
    o_ref[...] = y


def _matmul(x, w, b, act=None, bn=512):
    n, k = x.shape
    m = w.shape[0]
    return pl.pallas_call(
        functools.partial(_mm_body, act=act),
        grid=(n // bn,),
        in_specs=[pl.BlockSpec((bn, k), lambda i: (i, 0)),
                  pl.BlockSpec((m, k), lambda i: (0, 0)),
                  pl.BlockSpec((1, m), lambda i: (0, 0))],
        out_specs=pl.BlockSpec((bn, m), lambda i: (i, 0)),
        out_shape=jax.ShapeDtypeStruct((n, m), jnp.float32),
    )(x, w, b.reshape(1, -1))


def _combine_body(p_ref, skip_ref, cur_ref, w1_ref, w2_ref, bs_ref, o_ref,
                  *, cp, kind):
    p0 = p_ref[0]
    p1 = p_ref[1]
    h = 0.5 * (p0[:, :cp] + p0[:, cp:] + p1[:, :cp] + p1[:, cp:])
    h = h + skip_ref[...]
    if kind == 0:
        o_ref[...] = jnp.maximum(h, 0.0)
        return
    if kind == 1:
        h = jnp.maximum(h, 0.0)
    cur = cur_ref[...]
    z = lax.dot_general(h, w1_ref[...], (((1,), (1,)), ((), ())),
                        preferred_element_type=jnp.float32)
    z = z + lax.dot_general(cur, w2_ref[...], (((1,), (1,)), ((), ())),
                            preferred_element_type=jnp.float32)
    z = jax.nn.sigmoid(z + bs_ref[...])
    o_ref[...] = z * h + (1.0 - z) * cur


def _combine(p, skip, cur, w1, w2, bs, cp, kind, bn=512):
    d = 2 * cp
    return pl.pallas_call(
        functools.partial(_combine_body, cp=cp, kind=kind),
        grid=(NP // bn,),
        in_specs=[pl.BlockSpec((2, bn, d), lambda i: (0, i, 0)),
                  pl.BlockSpec((bn, cp), lambda i: (i, 0)),
                  pl.BlockSpec((bn, cp), lambda i: (i, 0)),
                  pl.BlockSpec((cp, cp), lambda i: (0, 0)),
                  pl.BlockSpec((cp, cp), lambda i: (0, 0)),
                  pl.BlockSpec((1, cp), lambda i: (0, 0))],
        out_specs=pl.BlockSpec((bn, cp), lambda i: (i, 0)),
        out_shape=jax.ShapeDtypeStruct((NP, cp), jnp.float32),
    )(p, skip, cur, w1, w2, bs)


def _seg_rmw(accbuf, stagei, stagef, key16, v0, v1, is_max):
    lane = lax.iota(jnp.int32, 16)
    z16 = jnp.zeros((16,), jnp.int32)
    sk, perm = plsc.sort_key_val(key16, lane)
    stagei[0, pl.ds(0, 16)] = sk
    stagef[0, pl.ds(0, 16)] = v0
    stagef[1, pl.ds(0, 16)] = v1
    x0 = plsc.load_gather(stagef, [z16, perm])
    x1 = plsc.load_gather(stagef, [z16 + 1, perm])
    for s in (1, 2, 4, 8):
        idxs = jnp.maximum(lane - s, 0)
        kprev = plsc.load_gather(stagei, [z16, idxs])
        valid = jnp.logical_and(lane >= s, kprev == sk)
        stagef[0, pl.ds(0, 16)] = x0
        stagef[1, pl.ds(0, 16)] = x1
        p0 = plsc.load_gather(stagef, [z16, idxs])
        p1 = plsc.load_gather(stagef, [z16 + 1, idxs])
        if is_max:
            x0 = jnp.where(valid, jnp.maximum(x0, p0), x0)
            x1 = jnp.where(valid, jnp.maximum(x1, p1), x1)
        else:
            x0 = jnp.where(valid, x0 + p0, x0)
            x1 = jnp.where(valid, x1 + p1, x1)
    knext = plsc.load_gather(stagei, [z16, jnp.minimum(lane + 1, 15)])
    is_end = jnp.logical_or(lane == 15, knext != sk)
    for h, xv in ((0, x0), (1, x1)):
        ia = sk * 2 + h
        cur = plsc.load_gather(accbuf, [ia])
        newv = jnp.maximum(cur, xv) if is_max else cur + xv
        plsc.store_scatter(accbuf, [ia], newv, mask=is_end)


def _k1_alpha(d, cp, inv_sqrt_c):
    def body(q_hbm, k_hbm, src_hbm, dst_hbm, alpha_hbm, pamax_hbm,
             srcbuf, dstbuf, didx, sidx, alphabuf, amaxbuf, qrows, krows,
             stagei, stagef, sem_q, sem_k):
        w = _wid()
        ebase = w * EPT
        pltpu.sync_copy(src_hbm.at[pl.ds(ebase, EPT)], srcbuf)
        pltpu.sync_copy(dst_hbm.at[pl.ds(ebase, EPT)], dstbuf)

        def init(i, _):
            amaxbuf[pl.ds(i * 16, 16)] = jnp.full((16,), NEG, jnp.float32)
            return 0
        lax.fori_loop(0, (2 * NP) // 16, init, 0)

        lane = lax.iota(jnp.int32, 16)

        def group(g, _):
            didx[...] = dstbuf[pl.ds(g * 16, 16)]
            sidx[...] = srcbuf[pl.ds(g * 16, 16)]
            cq = pltpu.async_copy(q_hbm.at[didx], qrows, sem_q)
            ck = pltpu.async_copy(k_hbm.at[sidx], krows, sem_k)
            cq.wait()
            ck.wait()

            def feat(j, accs):
                a0, a1 = accs
                c0 = jnp.full((16,), 0, jnp.int32) + j
                q0 = plsc.load_gather(qrows, [lane, c0])
                k0 = plsc.load_gather(krows, [lane, c0])
                c1 = c0 + cp
                q1 = plsc.load_gather(qrows, [lane, c1])
                k1 = plsc.load_gather(krows, [lane, c1])
                return (a0 + q0 * k0, a1 + q1 * k1)
            zf = jnp.zeros((16,), jnp.float32)
            a0, a1 = lax.fori_loop(0, cp, feat, (zf, zf), unroll=8)
            a0 = a0 * inv_sqrt_c
            a1 = a1 * inv_sqrt_c
            alphabuf[0, pl.ds(g * 16, 16)] = a0
            alphabuf[1, pl.ds(g * 16, 16)] = a1
            _seg_rmw(amaxbuf, stagei, stagef, didx[...], a0, a1, True)
            return 0
        lax.fori_loop(0, GRP, group, 0)

        pltpu.sync_copy(alphabuf, alpha_hbm.at[w])
        pltpu.sync_copy(amaxbuf, pamax_hbm.at[w])

    return pl.kernel(
        body,
        compiler_params=pltpu.CompilerParams(use_tc_tiling_on_sc=False, needs_layout_passes=False),
        out_type=(jax.ShapeDtypeStruct((NW, 2, EPT), jnp.float32),
                  jax.ShapeDtypeStruct((NW, 2 * NP), jnp.float32)),
        mesh=_mesh,
        scratch_types=[
            pltpu.VMEM((EPT,), jnp.int32),
            pltpu.VMEM((EPT,), jnp.int32),
            pltpu.VMEM((16,), jnp.int32),
            pltpu.VMEM((16,), jnp.int32),
            pltpu.VMEM((2, EPT), jnp.float32),
            pltpu.VMEM((2 * NP,), jnp.float32),
            pltpu.VMEM((16, d), jnp.float32),
            pltpu.VMEM((16, d), jnp.float32),
            pltpu.VMEM((1, 16), jnp.int32),
            pltpu.VMEM((2, 16), jnp.float32),
            pltpu.SemaphoreType.DMA,
            pltpu.SemaphoreType.DMA,
        ],
    )


def _k_reduce(is_max):
    seg = (2 * NP) // NW

    def body(part_hbm, out_hbm, accbuf, tmpbuf, sem):
        w = _wid()
        nbase = w * seg
        pltpu.sync_copy(part_hbm.at[0, pl.ds(nbase, seg)], accbuf)

        def red(p, _):
            pltpu.async_copy(part_hbm.at[p, pl.ds(nbase, seg)], tmpbuf,
                             sem).wait()

            def vec(j, _):
                a = accbuf[pl.ds(j * 16, 16)]
                t = tmpbuf[pl.ds(j * 16, 16)]
                accbuf[pl.ds(j * 16, 16)] = (
                    jnp.maximum(a, t) if is_max else a + t)
                return 0
            lax.fori_loop(0, seg // 16, vec, 0)
            return 0
        lax.fori_loop(1, NW, red, 0)
        pltpu.sync_copy(accbuf, out_hbm.at[pl.ds(nbase, seg)])

    return pl.kernel(
        body,
        compiler_params=pltpu.CompilerParams(use_tc_tiling_on_sc=False, needs_layout_passes=False),
        out_type=jax.ShapeDtypeStruct((2 * NP,), jnp.float32),
        mesh=_mesh,
        scratch_types=[
            pltpu.VMEM((seg,), jnp.float32),
            pltpu.VMEM((seg,), jnp.float32),
            pltpu.SemaphoreType.DMA,
        ],
    )


def _k3_exp(cp):
    def body(alpha_hbm, dst_hbm, amax_hbm, ex_hbm, pden_hbm,
             alphabuf, dstbuf, amaxbuf, exbuf, denbuf, stagei, stagef):
        w = _wid()
        ebase = w * EPT
        pltpu.sync_copy(alpha_hbm.at[w], alphabuf)
        pltpu.sync_copy(dst_hbm.at[pl.ds(ebase, EPT)], dstbuf)
        pltpu.sync_copy(amax_hbm, amaxbuf)

        def zero(i, _):
            denbuf[pl.ds(i * 16, 16)] = jnp.zeros((16,), jnp.float32)
            return 0
        lax.fori_loop(0, (2 * NP) // 16, zero, 0)

        def group(g, _):
            dst16 = dstbuf[pl.ds(g * 16, 16)]
            exs = []
            for h in range(2):
                am = plsc.load_gather(amaxbuf, [dst16 * 2 + h])
                ex = jnp.exp(alphabuf[h, pl.ds(g * 16, 16)] - am)
                exbuf[h, pl.ds(g * 16, 16)] = ex
                exs.append(ex)
            _seg_rmw(denbuf, stagei, stagef, dst16, exs[0], exs[1], False)
            return 0
        lax.fori_loop(0, GRP, group, 0)

        pltpu.sync_copy(exbuf, ex_hbm.at[w])
        pltpu.sync_copy(denbuf, pden_hbm.at[w])

    return pl.kernel(
        body,
        compiler_params=pltpu.CompilerParams(use_tc_tiling_on_sc=False, needs_layout_passes=False),
        out_type=(jax.ShapeDtypeStruct((NW, 2, EPT), jnp.float32),
                  jax.ShapeDtypeStruct((NW, 2 * NP), jnp.float32)),
        mesh=_mesh,
        scratch_types=[
            pltpu.VMEM((2, EPT), jnp.float32),
            pltpu.VMEM((EPT,), jnp.int32),
            pltpu.VMEM((2 * NP,), jnp.float32),
            pltpu.VMEM((2, EPT), jnp.float32),
            pltpu.VMEM((2 * NP,), jnp.float32),
            pltpu.VMEM((1, 16), jnp.int32),
            pltpu.VMEM((2, 16), jnp.float32),
        ],
    )


def _k5_scatter(d, cp):
    binw = EPT + 16
    nch = cp // 16
    rows_per_tile = CH // 16

    def body(v_hbm, ex_hbm, src_hbm, dst_hbm, den_hbm, outp_hbm,
             exbuf, srcbuf, dstbuf, denbuf, bins,
             sidx, rowidx, vrows, zerobuf, shared, sem):
        sc = lax.axis_index("c")
        tile = lax.axis_index("s")
        w = tile * 2 + sc
        ebase = w * EPT
        pltpu.sync_copy(ex_hbm.at[w], exbuf.at[:, pl.ds(0, EPT)])
        pltpu.sync_copy(src_hbm.at[pl.ds(ebase, EPT)],
                        srcbuf.at[pl.ds(0, EPT)])
        pltpu.sync_copy(dst_hbm.at[pl.ds(ebase, EPT)],
                        dstbuf.at[pl.ds(0, EPT)])
        pltpu.sync_copy(den_hbm, denbuf)

        def group(g, _):
            dst16 = dstbuf[pl.ds(g * 16, 16)]
            for h in range(2):
                dn = plsc.load_gather(denbuf, [dst16 * 2 + h])
                exbuf[h, pl.ds(g * 16, 16)] = (
                    exbuf[h, pl.ds(g * 16, 16)] / (dn + 1e-16))
            return 0
        lax.fori_loop(0, GRP, group, 0)

        lane = lax.iota(jnp.int32, 16)
        exbuf[0, pl.ds(EPT, 16)] = jnp.zeros((16,), jnp.float32)
        exbuf[1, pl.ds(EPT, 16)] = jnp.zeros((16,), jnp.float32)
        srcbuf[pl.ds(EPT, 16)] = jnp.zeros((16,), jnp.int32)
        dstbuf[pl.ds(EPT, 16)] = jnp.zeros((16,), jnp.int32)

        def zb(i, _):
            for j in range(nch * 2):
                zerobuf[i, pl.ds(j * 16, 16)] = jnp.zeros((16,), jnp.float32)
            return 0
        lax.fori_loop(0, 8, zb, 0)

        for c in range(NCHUNK):
            for t in range(rows_per_tile // 8):
                pltpu.sync_copy(
                    zerobuf,
                    shared.at[pl.ds(tile * rows_per_tile + t * 8, 8), :])

            def binit(g, cnt):
                dst16 = dstbuf[pl.ds(g * 16, 16)]
                m = lax.shift_right_logical(dst16, CH_SHIFT) == c
                plsc.store_compressed(bins.at[pl.ds(cnt, 16)],
                                      lane + g * 16, mask=m)
                return cnt + jnp.sum(jnp.where(m, 1, 0))
            cnt = lax.fori_loop(0, GRP, binit, jnp.int32(0))
            bins[pl.ds(cnt, 16)] = jnp.full((16,), EPT, jnp.int32)
            plsc.subcore_barrier()

            def grp(j, _):
                idx16 = bins[pl.ds(j * 16, 16)]
                sidx[...] = plsc.load_gather(srcbuf, [idx16])
                dst16 = plsc.load_gather(dstbuf, [idx16])
                row16 = dst16 - c * CH
                row16 = jnp.clip(row16, 0, CH - 1)
                rowidx[...] = row16
                h0 = jnp.zeros((16,), jnp.int32)
                a0 = plsc.load_gather(exbuf, [h0, idx16])
                a1 = plsc.load_gather(exbuf, [h0 + 1, idx16])
                pltpu.async_copy(v_hbm.at[sidx], vrows, sem).wait()

                def col0(jf, _):
                    cj = h0 + jf
                    vc = plsc.load_gather(vrows, [lane, cj])
                    plsc.store_scatter(vrows, [lane, cj], vc * a0)
                    return 0
                lax.fori_loop(0, cp, col0, 0, unroll=8)

                def col1(jf, _):
                    cj = h0 + jf
                    vc = plsc.load_gather(vrows, [lane, cj])
                    plsc.store_scatter(vrows, [lane, cj], vc * a1)
                    return 0
                lax.fori_loop(cp, 2 * cp, col1, 0, unroll=8)
                pltpu.sync_copy(vrows, shared.at[rowidx], add=True)
                return 0
            lax.fori_loop(0, (cnt + 15) // 16, grp, 0)
            plsc.subcore_barrier()

            for t in range(rows_per_tile // 8):
                r = tile * rows_per_tile + t * 8
                pltpu.sync_copy(
                    shared.at[pl.ds(r, 8), :],
                    outp_hbm.at[sc, pl.ds(c * CH + r, 8), :])
            plsc.subcore_barrier()

    return pl.kernel(
        body,
        compiler_params=pltpu.CompilerParams(use_tc_tiling_on_sc=False, needs_layout_passes=False),
        out_type=jax.ShapeDtypeStruct((2, NP, d), jnp.float32),
        mesh=_mesh,
        scratch_types=[
            pltpu.VMEM((2, binw), jnp.float32),
            pltpu.VMEM((binw,), jnp.int32),
            pltpu.VMEM((binw,), jnp.int32),
            pltpu.VMEM((2 * NP,), jnp.float32),
            pltpu.VMEM((binw,), jnp.int32),
            pltpu.VMEM((16,), jnp.int32),
            pltpu.VMEM((16,), jnp.int32),
            pltpu.VMEM((16, d), jnp.float32),
            pltpu.VMEM((8, d), jnp.float32),
            pltpu.VMEM_SHARED((CH, d), jnp.float32),
            pltpu.SemaphoreType.DMA,
        ],
    )


def _k6_pool(cp):
    gpw = B // NW
    nch = cp // 16

    def body(cur_hbm, batch_hbm, pooled_hbm, batchbuf, rowbuf, poolbuf, sem):
        w = _wid()
        g0 = w * gpw
        pltpu.sync_copy(batch_hbm, batchbuf)

        def count(i, lohi):
            b16 = batchbuf[pl.ds(i * 16, 16)]
            lo = lohi[0] + jnp.sum(jnp.where(b16 < g0, 1, 0))
            hi = lohi[1] + jnp.sum(jnp.where(b16 < g0 + gpw, 1, 0))
            return (lo, hi)
        lo, hi = lax.fori_loop(0, NP // 16, count,
                               (jnp.int32(0), jnp.int32(0)))

        def init(g, _):
            for j in range(nch):
                poolbuf[g, pl.ds(j * 16, 16)] = jnp.full((16,), NEG,
                                                         jnp.float32)
            return 0
        lax.fori_loop(0, gpw, init, 0)

        nblk = (hi - lo + 31) // 32

        def blk(t, _):
            base = lo + t * 32
            pltpu.async_copy(cur_hbm.at[pl.ds(base, 32), :], rowbuf,
                             sem).wait()

            def row(r, _):
                i = base + r

                @pl.when(i < hi)
                def _():
                    g = batchbuf[pl.ds(i, 16)][0] - g0
                    for j in range(nch):
                        a = poolbuf[g, pl.ds(j * 16, 16)]
                        b = rowbuf[r, pl.ds(j * 16, 16)]
                        poolbuf[g, pl.ds(j * 16, 16)] = jnp.maximum(a, b)
                return 0
            lax.fori_loop(0, 32, row, 0)
            return 0
        lax.fori_loop(0, nblk, blk, 0)

        def fix(g, _):
            for j in range(nch):
                v = poolbuf[g, pl.ds(j * 16, 16)]
                poolbuf[g, pl.ds(j * 16, 16)] = jnp.where(
                    v > -1e29, v, 0.0)
            return 0
        lax.fori_loop(0, gpw, fix, 0)
        pltpu.sync_copy(poolbuf, pooled_hbm.at[pl.ds(g0, gpw), :])

    return pl.kernel(
        body,
        compiler_params=pltpu.CompilerParams(use_tc_tiling_on_sc=False, needs_layout_passes=False),
        out_type=jax.ShapeDtypeStruct((B, cp), jnp.float32),
        mesh=_mesh,
        scratch_types=[
            pltpu.VMEM((NP,), jnp.int32),
            pltpu.VMEM((32, cp), jnp.float32),
            pltpu.VMEM((gpw, cp), jnp.float32),
            pltpu.SemaphoreType.DMA,
        ],
    )


def _head_body(x_ref, xq_ref, wx1_ref, bx1_ref, wx2_ref,
               wq1_ref, bq1_ref, wq2_ref,
               f1w_ref, f1b_ref, f2w_ref, f2b_ref, ow_ref, ob_ref, o_ref):
    def lin(v, w_ref, b=None):
        y = lax.dot_general(v, w_ref[...], (((1,), (1,)), ((), ())),
                            preferred_element_type=jnp.float32)
        return y if b is None else y + b
    x = x_ref[...]
    xq = xq_ref[...]
    ax = lin(jnp.tanh(lin(x, wx1_ref, bx1_ref[...])), wx2_ref)
    axq = lin(jnp.tanh(lin(xq, wq1_ref, bq1_ref[...])), wq2_ref)
    emb = jnp.concatenate([ax * x, axq * xq], axis=1)
    h = jnp.maximum(lin(emb, f1w_ref, f1b_ref[...]), 0.0)
    h = jnp.maximum(lin(h, f2w_ref, f2b_ref[...]), 0.0)
    o_ref[...] = lin(h, ow_ref, ob_ref[...])


def _head(x, xq, p):
    ow = jnp.pad(p["out"][0], ((0, 7), (0, 0)))
    ob = jnp.pad(p["out"][1], ((0, 7),)).reshape(1, 8)
    args = (x, xq, p["att_x1"][0], p["att_x1"][1].reshape(1, -1),
            p["att_x2"][0], p["att_q1"][0], p["att_q1"][1].reshape(1, -1),
            p["att_q2"][0], p["fc1"][0], p["fc1"][1].reshape(1, -1),
            p["fc2"][0], p["fc2"][1].reshape(1, -1), ow, ob)
    out = pl.pallas_call(
        _head_body,
        out_shape=jax.ShapeDtypeStruct((B, 8), jnp.float32),
    )(*args)
    return out[:, :1]


def _pad_conv_params(p, c, cp, fin, fp):
    ws = []
    bs = []
    for name in ("q", "k", "v"):
        wt, bt = p[name]
        w2 = wt.reshape(2, c, fin)
        w2 = jnp.pad(w2, ((0, 0), (0, cp - c), (0, fp - fin)))
        ws.append(w2.reshape(2 * cp, fp))
        bs.append(jnp.pad(bt.reshape(2, c), ((0, 0), (0, cp - c))).reshape(-1))
    ws.append(jnp.pad(p["s"][0], ((0, cp - c), (0, fp - fin))))
    bs.append(jnp.pad(p["s"][1], ((0, cp - c),)))
    return jnp.concatenate(ws, axis=0), jnp.concatenate(bs, axis=0)


def _branch(x, edge_index, batch, convs, fc1, fc2, bias, g1, g2, c, cp, f):
    d = 2 * cp
    fp = ((f + 15) // 16) * 16
    inv_sqrt_c = float(1.0 / (c ** 0.5))

    xp = jnp.pad(x, ((0, NP - N_NODE), (0, fp - f)))
    src = jnp.concatenate(
        [edge_index[0], jnp.zeros((E_PAD - E,), jnp.int32)])
    dst = jnp.concatenate(
        [edge_index[1], jnp.full((E_PAD - E,), DUMMY, jnp.int32)])
    batchp = jnp.concatenate(
        [batch, jnp.full((NP - N_NODE,), B, jnp.int32)])

    k1 = _k1_alpha(d, cp, inv_sqrt_c)
    kmax = _k_reduce(True)
    ksum = _k_reduce(False)
    k3 = _k3_exp(cp)
    k5 = _k5_scatter(d, cp)

    w1p = jnp.pad(fc1[0], ((0, cp - c), (0, cp - c)))
    w2p = jnp.pad(fc2[0], ((0, cp - c), (0, cp - c)))
    bsum = jnp.pad(fc1[1] + fc2[1] + bias[0], ((0, cp - c),)).reshape(1, cp)

    cur = xp
    fin = fp
    for i, p in enumerate(convs):
        wcat, bcat = _pad_conv_params(p, c, cp, [f, c, c][i], fin)
        y = _matmul(cur, wcat, bcat)
        q = y[:, :d]
        k = y[:, d:2 * d]
        v = y[:, 2 * d:3 * d]
        skip = y[:, 3 * d:]
        alpha, pamax = k1(q, k, src, dst)
        amax = kmax(pamax)
        ex, pden = k3(alpha, dst, amax)
        den = ksum(pden)
        outp = k5(v, ex, src, dst, den)
        kind = min(i, 2)
        cur = _combine(outp, skip, skip if i == 0 else cur, w1p, w2p, bsum,
                       cp, kind)
        fin = cp

    pooled = _k6_pool(cp)(cur, batchp)
    g1w = jnp.pad(g1[0], ((0, 0), (0, cp - c)))
    g = _matmul(pooled, g1w, g1[1], act="relu")
    return _matmul(g, g2[0], g2[1])


def kernel(mol_x, mol_edge_index, mol_batch, clique_x, clique_edge_index,
           clique_batch, data_pre, params):
    p = params
    x = _branch(mol_x, mol_edge_index, mol_batch, p["mol_convs"],
                p["mol_seq_fc1"], p["mol_seq_fc2"], p["mol_bias"],
                p["mol_g1"], p["mol_g2"], 312, 320, 78)
    xq = _branch(clique_x, clique_edge_index, clique_batch, p["cli_convs"],
                 p["cli_seq_fc1"], p["cli_seq_fc2"], p["cli_bias"],
                 p["cli_g1"], p["cli_g2"], 368, 368, 92)
    return _head(x, xq, p)


# speedup vs baseline: 10.3039x; 4.1665x over previous
import functools

import jax
import jax.numpy as jnp
from jax import lax
from jax.experimental import pallas as pl
from jax.experimental.pallas import tpu as pltpu
from jax.experimental.pallas import tpu_sc as plsc

N_NODE = 10000
NP = 10240
E = 160000
NW = 32
EPT = 5008
E_PAD = NW * EPT
GRP = EPT // 16
B = 512
CH = 512
NCHUNK = NP // CH
CH_SHIFT = 9
DUMMY = NP - 8
NEG = -1e30

_mesh = plsc.VectorSubcoreMesh(core_axis_name="c", subcore_axis_name="s")


def _wid():
    return lax.axis_index("s") * 2 + lax.axis_index("c")


def _mm_body(x_ref, w_ref, b_ref, o_ref, *, act):
    y = lax.dot_general(x_ref[...], w_ref[...], (((1,), (1,)), ((), ())),
                        preferred_element_type=jnp.float32)
    y = y + b_ref[...]
    if act == "relu":
        y = jnp.maximum(y, 0.0)
    o_ref[...] = y


def _matmul(x, w, b, act=None, bn=512):
    n, k = x.shape
    m = w.shape[0]
    return pl.pallas_call(
        functools.partial(_mm_body, act=act),
        grid=(n // bn,),
        in_specs=[pl.BlockSpec((bn, k), lambda i: (i, 0)),
                  pl.BlockSpec((m, k), lambda i: (0, 0)),
                  pl.BlockSpec((1, m), lambda i: (0, 0))],
        out_specs=pl.BlockSpec((bn, m), lambda i: (i, 0)),
        out_shape=jax.ShapeDtypeStruct((n, m), jnp.float32),
    )(x, w, b.reshape(1, -1))


def _combine_body(p_ref, skip_ref, cur_ref, w1_ref, w2_ref, bs_ref, o_ref,
                  *, cp, kind):
    p0 = p_ref[0]
    p1 = p_ref[1]
    h = 0.5 * (p0[:, :cp] + p0[:, cp:] + p1[:, :cp] + p1[:, cp:])
    h = h + skip_ref[...]
    if kind == 0:
        o_ref[...] = jnp.maximum(h, 0.0)
        return
    if kind == 1:
        h = jnp.maximum(h, 0.0)
    cur = cur_ref[...]
    z = lax.dot_general(h, w1_ref[...], (((1,), (1,)), ((), ())),
                        preferred_element_type=jnp.float32)
    z = z + lax.dot_general(cur, w2_ref[...], (((1,), (1,)), ((), ())),
                            preferred_element_type=jnp.float32)
    z = jax.nn.sigmoid(z + bs_ref[...])
    o_ref[...] = z * h + (1.0 - z) * cur


def _combine(p, skip, cur, w1, w2, bs, cp, kind, bn=512):
    d = 2 * cp
    return pl.pallas_call(
        functools.partial(_combine_body, cp=cp, kind=kind),
        grid=(NP // bn,),
        in_specs=[pl.BlockSpec((2, bn, d), lambda i: (0, i, 0)),
                  pl.BlockSpec((bn, cp), lambda i: (i, 0)),
                  pl.BlockSpec((bn, cp), lambda i: (i, 0)),
                  pl.BlockSpec((cp, cp), lambda i: (0, 0)),
                  pl.BlockSpec((cp, cp), lambda i: (0, 0)),
                  pl.BlockSpec((1, cp), lambda i: (0, 0))],
        out_specs=pl.BlockSpec((bn, cp), lambda i: (i, 0)),
        out_shape=jax.ShapeDtypeStruct((NP, cp), jnp.float32),
    )(p, skip, cur, w1, w2, bs)


def _seg_rmw(accbuf, stagei, stagef, key16, v0, v1, is_max):
    lane = lax.iota(jnp.int32, 16)
    z16 = jnp.zeros((16,), jnp.int32)
    sk, perm = plsc.sort_key_val(key16, lane)
    stagei[0, pl.ds(0, 16)] = sk
    stagef[0, pl.ds(0, 16)] = v0
    stagef[1, pl.ds(0, 16)] = v1
    x0 = plsc.load_gather(stagef, [z16, perm])
    x1 = plsc.load_gather(stagef, [z16 + 1, perm])
    for s in (1, 2, 4, 8):
        idxs = jnp.maximum(lane - s, 0)
        kprev = plsc.load_gather(stagei, [z16, idxs])
        valid = jnp.logical_and(lane >= s, kprev == sk)
        stagef[0, pl.ds(0, 16)] = x0
        stagef[1, pl.ds(0, 16)] = x1
        p0 = plsc.load_gather(stagef, [z16, idxs])
        p1 = plsc.load_gather(stagef, [z16 + 1, idxs])
        if is_max:
            x0 = jnp.where(valid, jnp.maximum(x0, p0), x0)
            x1 = jnp.where(valid, jnp.maximum(x1, p1), x1)
        else:
            x0 = jnp.where(valid, x0 + p0, x0)
            x1 = jnp.where(valid, x1 + p1, x1)
    knext = plsc.load_gather(stagei, [z16, jnp.minimum(lane + 1, 15)])
    is_end = jnp.logical_or(lane == 15, knext != sk)
    for h, xv in ((0, x0), (1, x1)):
        ia = sk * 2 + h
        cur = plsc.load_gather(accbuf, [ia])
        newv = jnp.maximum(cur, xv) if is_max else cur + xv
        plsc.store_scatter(accbuf, [ia], newv, mask=is_end)


def _k1_alpha(d, cp, inv_sqrt_c):
    nch = cp // 16

    def body(q_hbm, k_hbm, src_hbm, dst_hbm, alpha_hbm, pamax_hbm,
             srcbuf, dstbuf, di0, si0, di1, si1, alphabuf, amaxbuf,
             qr0, kr0, qr1, kr1, dst0, dst1,
             stagei, stagef, sq0, sk0, sq1, sk1):
        w = _wid()
        ebase = w * EPT
        pltpu.sync_copy(src_hbm.at[pl.ds(ebase, EPT)], srcbuf)
        pltpu.sync_copy(dst_hbm.at[pl.ds(ebase, EPT)], dstbuf)

        dis = [di0, di1]
        sis = [si0, si1]
        qrs = [qr0, qr1]
        krs = [kr0, kr1]
        sqs = [sq0, sq1]
        sks = [sk0, sk1]

        def init(i, _):
            amaxbuf[pl.ds(i * 16, 16)] = jnp.full((16,), NEG, jnp.float32)
            return 0
        lax.fori_loop(0, (2 * NP) // 16, init, 0)

        lane = lax.iota(jnp.int32, 16)
        z16 = jnp.zeros((16,), jnp.int32)
        zf = jnp.zeros((16,), jnp.float32)

        def issue(g, b):
            dis[b][...] = dstbuf[pl.ds(g * 16, 16)]
            sis[b][...] = srcbuf[pl.ds(g * 16, 16)]
            pltpu.async_copy(q_hbm.at[dis[b]], qrs[b], sqs[b])
            pltpu.async_copy(k_hbm.at[sis[b]], krs[b], sks[b])

        def waitb(b):
            pltpu.make_async_copy(q_hbm.at[dis[b]], qrs[b], sqs[b]).wait()
            pltpu.make_async_copy(k_hbm.at[sis[b]], krs[b], sks[b]).wait()

        def compute(g, b):
            qr = qrs[b]
            kr = krs[b]
            for r in range(16):
                def c0(j, acc, r=r, qr=qr, kr=kr):
                    return acc + (qr[r, pl.ds(j * 16, 16)]
                                  * kr[r, pl.ds(j * 16, 16)])
                s0 = lax.fori_loop(0, nch, c0, zf, unroll=4)

                def c1(j, acc, r=r, qr=qr, kr=kr):
                    return acc + (qr[r, pl.ds(cp + j * 16, 16)]
                                  * kr[r, pl.ds(cp + j * 16, 16)])
                s1 = lax.fori_loop(0, nch, c1, zf, unroll=4)
                dst0[r, pl.ds(0, 16)] = s0
                dst1[r, pl.ds(0, 16)] = s1
            a0 = zf
            a1 = zf
            for cc in range(16):
                a0 = a0 + plsc.load_gather(dst0, [lane, z16 + cc])
                a1 = a1 + plsc.load_gather(dst1, [lane, z16 + cc])
            a0 = a0 * inv_sqrt_c
            a1 = a1 * inv_sqrt_c
            alphabuf[0, pl.ds(g * 16, 16)] = a0
            alphabuf[1, pl.ds(g * 16, 16)] = a1
            d16 = dstbuf[pl.ds(g * 16, 16)]
            _seg_rmw(amaxbuf, stagei, stagef, d16, a0, a1, True)

        issue(0, 0)

        def pair(p, _):
            g = p * 2
            for b in range(2):
                gg = g + b

                @pl.when(gg < GRP)
                def _(gg=gg, b=b):
                    waitb(b)

                    @pl.when(gg + 1 < GRP)
                    def _():
                        issue(gg + 1, 1 - b)
                    compute(gg, b)
            return 0
        lax.fori_loop(0, (GRP + 1) // 2, pair, 0)

        pltpu.sync_copy(alphabuf, alpha_hbm.at[w])
        pltpu.sync_copy(amaxbuf, pamax_hbm.at[w])

    return pl.kernel(
        body,
        compiler_params=pltpu.CompilerParams(use_tc_tiling_on_sc=False, needs_layout_passes=False),
        out_type=(jax.ShapeDtypeStruct((NW, 2, EPT), jnp.float32),
                  jax.ShapeDtypeStruct((NW, 2 * NP), jnp.float32)),
        mesh=_mesh,
        scratch_types=[
            pltpu.VMEM((EPT,), jnp.int32),
            pltpu.VMEM((EPT,), jnp.int32),
            pltpu.VMEM((16,), jnp.int32),
            pltpu.VMEM((16,), jnp.int32),
            pltpu.VMEM((16,), jnp.int32),
            pltpu.VMEM((16,), jnp.int32),
            pltpu.VMEM((2, EPT), jnp.float32),
            pltpu.VMEM((2 * NP,), jnp.float32),
            pltpu.VMEM((16, d), jnp.float32),
            pltpu.VMEM((16, d), jnp.float32),
            pltpu.VMEM((16, d), jnp.float32),
            pltpu.VMEM((16, d), jnp.float32),
            pltpu.VMEM((16, 17), jnp.float32),
            pltpu.VMEM((16, 17), jnp.float32),
            pltpu.VMEM((1, 16), jnp.int32),
            pltpu.VMEM((2, 16), jnp.float32),
            pltpu.SemaphoreType.DMA,
            pltpu.SemaphoreType.DMA,
            pltpu.SemaphoreType.DMA,
            pltpu.SemaphoreType.DMA,
        ],
    )


def _k_reduce(is_max):
    seg = (2 * NP) // NW

    def body(part_hbm, out_hbm, accbuf, tmpbuf, sem):
        w = _wid()
        nbase = w * seg
        pltpu.sync_copy(part_hbm.at[0, pl.ds(nbase, seg)], accbuf)

        def red(p, _):
            pltpu.async_copy(part_hbm.at[p, pl.ds(nbase, seg)], tmpbuf,
                             sem).wait()

            def vec(j, _):
                a = accbuf[pl.ds(j * 16, 16)]
                t = tmpbuf[pl.ds(j * 16, 16)]
                accbuf[pl.ds(j * 16, 16)] = (
                    jnp.maximum(a, t) if is_max else a + t)
                return 0
            lax.fori_loop(0, seg // 16, vec, 0)
            return 0
        lax.fori_loop(1, NW, red, 0)
        pltpu.sync_copy(accbuf, out_hbm.at[pl.ds(nbase, seg)])

    return pl.kernel(
        body,
        compiler_params=pltpu.CompilerParams(use_tc_tiling_on_sc=False, needs_layout_passes=False),
        out_type=jax.ShapeDtypeStruct((2 * NP,), jnp.float32),
        mesh=_mesh,
        scratch_types=[
            pltpu.VMEM((seg,), jnp.float32),
            pltpu.VMEM((seg,), jnp.float32),
            pltpu.SemaphoreType.DMA,
        ],
    )


def _k3_exp(cp):
    def body(alpha_hbm, dst_hbm, amax_hbm, ex_hbm, pden_hbm,
             alphabuf, dstbuf, amaxbuf, exbuf, denbuf, stagei, stagef):
        w = _wid()
        ebase = w * EPT
        pltpu.sync_copy(alpha_hbm.at[w], alphabuf)
        pltpu.sync_copy(dst_hbm.at[pl.ds(ebase, EPT)], dstbuf)
        pltpu.sync_copy(amax_hbm, amaxbuf)

        def zero(i, _):
            denbuf[pl.ds(i * 16, 16)] = jnp.zeros((16,), jnp.float32)
            return 0
        lax.fori_loop(0, (2 * NP) // 16, zero, 0)

        def group(g, _):
            dst16 = dstbuf[pl.ds(g * 16, 16)]
            exs = []
            for h in range(2):
                am = plsc.load_gather(amaxbuf, [dst16 * 2 + h])
                ex = jnp.exp(alphabuf[h, pl.ds(g * 16, 16)] - am)
                exbuf[h, pl.ds(g * 16, 16)] = ex
                exs.append(ex)
            _seg_rmw(denbuf, stagei, stagef, dst16, exs[0], exs[1], False)
            return 0
        lax.fori_loop(0, GRP, group, 0)

        pltpu.sync_copy(exbuf, ex_hbm.at[w])
        pltpu.sync_copy(denbuf, pden_hbm.at[w])

    return pl.kernel(
        body,
        compiler_params=pltpu.CompilerParams(use_tc_tiling_on_sc=False, needs_layout_passes=False),
        out_type=(jax.ShapeDtypeStruct((NW, 2, EPT), jnp.float32),
                  jax.ShapeDtypeStruct((NW, 2 * NP), jnp.float32)),
        mesh=_mesh,
        scratch_types=[
            pltpu.VMEM((2, EPT), jnp.float32),
            pltpu.VMEM((EPT,), jnp.int32),
            pltpu.VMEM((2 * NP,), jnp.float32),
            pltpu.VMEM((2, EPT), jnp.float32),
            pltpu.VMEM((2 * NP,), jnp.float32),
            pltpu.VMEM((1, 16), jnp.int32),
            pltpu.VMEM((2, 16), jnp.float32),
        ],
    )


def _k5_scatter(d, cp):
    binw = EPT + 16
    nch = cp // 16
    rows_per_tile = CH // 16

    def body(v_hbm, ex_hbm, src_hbm, dst_hbm, den_hbm, outp_hbm,
             exbuf, srcbuf, dstbuf, denbuf, bins,
             si0, si1, rowidx, vr0, vr1, astage, zerobuf, shared,
             sv0, sv1):
        sc = lax.axis_index("c")
        tile = lax.axis_index("s")
        w = tile * 2 + sc
        ebase = w * EPT
        pltpu.sync_copy(ex_hbm.at[w], exbuf.at[:, pl.ds(0, EPT)])
        pltpu.sync_copy(src_hbm.at[pl.ds(ebase, EPT)],
                        srcbuf.at[pl.ds(0, EPT)])
        pltpu.sync_copy(dst_hbm.at[pl.ds(ebase, EPT)],
                        dstbuf.at[pl.ds(0, EPT)])
        pltpu.sync_copy(den_hbm, denbuf)

        sis = [si0, si1]
        vrs = [vr0, vr1]
        svs = [sv0, sv1]

        def group(g, _):
            dst16 = dstbuf[pl.ds(g * 16, 16)]
            for h in range(2):
                dn = plsc.load_gather(denbuf, [dst16 * 2 + h])
                exbuf[h, pl.ds(g * 16, 16)] = (
                    exbuf[h, pl.ds(g * 16, 16)] / (dn + 1e-16))
            return 0
        lax.fori_loop(0, GRP, group, 0)

        lane = lax.iota(jnp.int32, 16)
        z16 = jnp.zeros((16,), jnp.int32)
        exbuf[0, pl.ds(EPT, 16)] = jnp.zeros((16,), jnp.float32)
        exbuf[1, pl.ds(EPT, 16)] = jnp.zeros((16,), jnp.float32)
        srcbuf[pl.ds(EPT, 16)] = jnp.zeros((16,), jnp.int32)
        dstbuf[pl.ds(EPT, 16)] = jnp.zeros((16,), jnp.int32)

        def zb(i, _):
            for j in range(nch * 2):
                zerobuf[i, pl.ds(j * 16, 16)] = jnp.zeros((16,), jnp.float32)
            return 0
        lax.fori_loop(0, 8, zb, 0)

        def issue5(j, b):
            idx16 = bins[pl.ds(j * 16, 16)]
            sis[b][...] = plsc.load_gather(srcbuf, [idx16])
            pltpu.async_copy(v_hbm.at[sis[b]], vrs[b], svs[b])

        def wait5(b):
            pltpu.make_async_copy(v_hbm.at[sis[b]], vrs[b], svs[b]).wait()

        def chunk(c, _):
            for t in range(rows_per_tile // 8):
                pltpu.sync_copy(
                    zerobuf,
                    shared.at[pl.ds(tile * rows_per_tile + t * 8, 8), :])

            def binit(g, cnt):
                dst16 = dstbuf[pl.ds(g * 16, 16)]
                m = lax.shift_right_logical(dst16, CH_SHIFT) == c
                plsc.store_compressed(bins.at[pl.ds(cnt, 16)],
                                      lane + g * 16, mask=m)
                return cnt + jnp.sum(jnp.where(m, 1, 0))
            cnt = lax.fori_loop(0, GRP, binit, jnp.int32(0))
            bins[pl.ds(cnt, 16)] = jnp.full((16,), EPT, jnp.int32)
            plsc.subcore_barrier()
            ngrp = (cnt + 15) // 16

            def compute5(j, b):
                vr = vrs[b]
                idx16 = bins[pl.ds(j * 16, 16)]
                dst16 = plsc.load_gather(dstbuf, [idx16])
                row16 = jnp.clip(dst16 - c * CH, 0, CH - 1)
                rowidx[...] = row16
                a0 = plsc.load_gather(exbuf, [z16, idx16])
                a1 = plsc.load_gather(exbuf, [z16 + 1, idx16])
                astage[0, pl.ds(0, 16)] = a0
                astage[1, pl.ds(0, 16)] = a1
                for r in range(16):
                    b0 = plsc.load_gather(astage, [z16, z16 + r])
                    b1 = plsc.load_gather(astage, [z16 + 1, z16 + r])

                    def s0(j2, _, r=r, vr=vr, b0=b0):
                        vr[r, pl.ds(j2 * 16, 16)] = (
                            vr[r, pl.ds(j2 * 16, 16)] * b0)
                        return 0
                    lax.fori_loop(0, nch, s0, 0, unroll=4)

                    def s1(j2, _, r=r, vr=vr, b1=b1):
                        vr[r, pl.ds(cp + j2 * 16, 16)] = (
                            vr[r, pl.ds(cp + j2 * 16, 16)] * b1)
                        return 0
                    lax.fori_loop(0, nch, s1, 0, unroll=4)
                pltpu.sync_copy(vr, shared.at[rowidx], add=True)

            @pl.when(ngrp > 0)
            def _():
                issue5(0, 0)

            def pair(p, _):
                for b in range(2):
                    gg = p * 2 + b

                    @pl.when(gg < ngrp)
                    def _(gg=gg, b=b):
                        wait5(b)

                        @pl.when(gg + 1 < ngrp)
                        def _():
                            issue5(gg + 1, 1 - b)
                        compute5(gg, b)
                return 0
            lax.fori_loop(0, (ngrp + 1) // 2, pair, 0)
            plsc.subcore_barrier()

            for t in range(rows_per_tile // 8):
                r = tile * rows_per_tile + t * 8
                pltpu.sync_copy(
                    shared.at[pl.ds(r, 8), :],
                    outp_hbm.at[sc, pl.ds(c * CH + r, 8), :])
            plsc.subcore_barrier()
            return 0
        lax.fori_loop(0, NCHUNK, chunk, 0)

    return pl.kernel(
        body,
        compiler_params=pltpu.CompilerParams(use_tc_tiling_on_sc=False, needs_layout_passes=False),
        out_type=jax.ShapeDtypeStruct((2, NP, d), jnp.float32),
        mesh=_mesh,
        scratch_types=[
            pltpu.VMEM((2, binw), jnp.float32),
            pltpu.VMEM((binw,), jnp.int32),
            pltpu.VMEM((binw,), jnp.int32),
            pltpu.VMEM((2 * NP,), jnp.float32),
            pltpu.VMEM((binw,), jnp.int32),
            pltpu.VMEM((16,), jnp.int32),
            pltpu.VMEM((16,), jnp.int32),
            pltpu.VMEM((16,), jnp.int32),
            pltpu.VMEM((16, d), jnp.float32),
            pltpu.VMEM((16, d), jnp.float32),
            pltpu.VMEM((2, 17), jnp.float32),
            pltpu.VMEM((8, d), jnp.float32),
            pltpu.VMEM_SHARED((CH, d), jnp.float32),
            pltpu.SemaphoreType.DMA,
            pltpu.SemaphoreType.DMA,
        ],
    )


def _k6_pool(cp):
    gpw = B // NW
    nch = cp // 16

    def body(cur_hbm, batch_hbm, pooled_hbm, batchbuf, rowbuf, poolbuf, sem):
        w = _wid()
        g0 = w * gpw
        pltpu.sync_copy(batch_hbm, batchbuf)

        def count(i, lohi):
            b16 = batchbuf[pl.ds(i * 16, 16)]
            lo = lohi[0] + jnp.sum(jnp.where(b16 < g0, 1, 0))
            hi = lohi[1] + jnp.sum(jnp.where(b16 < g0 + gpw, 1, 0))
            return (lo, hi)
        lo, hi = lax.fori_loop(0, NP // 16, count,
                               (jnp.int32(0), jnp.int32(0)))

        def init(g, _):
            for j in range(nch):
                poolbuf[g, pl.ds(j * 16, 16)] = jnp.full((16,), NEG,
                                                         jnp.float32)
            return 0
        lax.fori_loop(0, gpw, init, 0)

        nblk = (hi - lo + 31) // 32

        def blk(t, _):
            base = lo + t * 32
            pltpu.async_copy(cur_hbm.at[pl.ds(base, 32), :], rowbuf,
                             sem).wait()

            def row(r, _):
                i = base + r

                @pl.when(i < hi)
                def _():
                    g = batchbuf[pl.ds(i, 16)][0] - g0
                    for j in range(nch):
                        a = poolbuf[g, pl.ds(j * 16, 16)]
                        b = rowbuf[r, pl.ds(j * 16, 16)]
                        poolbuf[g, pl.ds(j * 16, 16)] = jnp.maximum(a, b)
                return 0
            lax.fori_loop(0, 32, row, 0)
            return 0
        lax.fori_loop(0, nblk, blk, 0)

        def fix(g, _):
            for j in range(nch):
                v = poolbuf[g, pl.ds(j * 16, 16)]
                poolbuf[g, pl.ds(j * 16, 16)] = jnp.where(
                    v > -1e29, v, 0.0)
            return 0
        lax.fori_loop(0, gpw, fix, 0)
        pltpu.sync_copy(poolbuf, pooled_hbm.at[pl.ds(g0, gpw), :])

    return pl.kernel(
        body,
        compiler_params=pltpu.CompilerParams(use_tc_tiling_on_sc=False, needs_layout_passes=False),
        out_type=jax.ShapeDtypeStruct((B, cp), jnp.float32),
        mesh=_mesh,
        scratch_types=[
            pltpu.VMEM((NP,), jnp.int32),
            pltpu.VMEM((32, cp), jnp.float32),
            pltpu.VMEM((gpw, cp), jnp.float32),
            pltpu.SemaphoreType.DMA,
        ],
    )


def _head_body(x_ref, xq_ref, wx1_ref, bx1_ref, wx2_ref,
               wq1_ref, bq1_ref, wq2_ref,
               f1w_ref, f1b_ref, f2w_ref, f2b_ref, ow_ref, ob_ref, o_ref):
    def lin(v, w_ref, b=None):
        y = lax.dot_general(v, w_ref[...], (((1,), (1,)), ((), ())),
                            preferred_element_type=jnp.float32)
        return y if b is None else y + b
    x = x_ref[...]
    xq = xq_ref[...]
    ax = lin(jnp.tanh(lin(x, wx1_ref, bx1_ref[...])), wx2_ref)
    axq = lin(jnp.tanh(lin(xq, wq1_ref, bq1_ref[...])), wq2_ref)
    emb = jnp.concatenate([ax * x, axq * xq], axis=1)
    h = jnp.maximum(lin(emb, f1w_ref, f1b_ref[...]), 0.0)
    h = jnp.maximum(lin(h, f2w_ref, f2b_ref[...]), 0.0)
    o_ref[...] = lin(h, ow_ref, ob_ref[...])


def _head(x, xq, p):
    ow = jnp.pad(p["out"][0], ((0, 7), (0, 0)))
    ob = jnp.pad(p["out"][1], ((0, 7),)).reshape(1, 8)
    args = (x, xq, p["att_x1"][0], p["att_x1"][1].reshape(1, -1),
            p["att_x2"][0], p["att_q1"][0], p["att_q1"][1].reshape(1, -1),
            p["att_q2"][0], p["fc1"][0], p["fc1"][1].reshape(1, -1),
            p["fc2"][0], p["fc2"][1].reshape(1, -1), ow, ob)
    out = pl.pallas_call(
        _head_body,
        out_shape=jax.ShapeDtypeStruct((B, 8), jnp.float32),
    )(*args)
    return out[:, :1]


def _pad_conv_params(p, c, cp, fin, fp):
    ws = []
    bs = []
    for name in ("q", "k", "v"):
        wt, bt = p[name]
        w2 = wt.reshape(2, c, fin)
        w2 = jnp.pad(w2, ((0, 0), (0, cp - c), (0, fp - fin)))
        ws.append(w2.reshape(2 * cp, fp))
        bs.append(jnp.pad(bt.reshape(2, c), ((0, 0), (0, cp - c))).reshape(-1))
    ws.append(jnp.pad(p["s"][0], ((0, cp - c), (0, fp - fin))))
    bs.append(jnp.pad(p["s"][1], ((0, cp - c),)))
    return jnp.concatenate(ws, axis=0), jnp.concatenate(bs, axis=0)


def _branch(x, edge_index, batch, convs, fc1, fc2, bias, g1, g2, c, cp, f):
    d = 2 * cp
    fp = ((f + 15) // 16) * 16
    inv_sqrt_c = float(1.0 / (c ** 0.5))

    xp = jnp.pad(x, ((0, NP - N_NODE), (0, fp - f)))
    src = jnp.concatenate(
        [edge_index[0], jnp.zeros((E_PAD - E,), jnp.int32)])
    dst = jnp.concatenate(
        [edge_index[1], jnp.full((E_PAD - E,), DUMMY, jnp.int32)])
    batchp = jnp.concatenate(
        [batch, jnp.full((NP - N_NODE,), B, jnp.int32)])

    k1 = _k1_alpha(d, cp, inv_sqrt_c)
    kmax = _k_reduce(True)
    ksum = _k_reduce(False)
    k3 = _k3_exp(cp)
    k5 = _k5_scatter(d, cp)

    w1p = jnp.pad(fc1[0], ((0, cp - c), (0, cp - c)))
    w2p = jnp.pad(fc2[0], ((0, cp - c), (0, cp - c)))
    bsum = jnp.pad(fc1[1] + fc2[1] + bias[0], ((0, cp - c),)).reshape(1, cp)

    cur = xp
    fin = fp
    for i, p in enumerate(convs):
        wcat, bcat = _pad_conv_params(p, c, cp, [f, c, c][i], fin)
        y = _matmul(cur, wcat, bcat)
        q = y[:, :d]
        k = y[:, d:2 * d]
        v = y[:, 2 * d:3 * d]
        skip = y[:, 3 * d:]
        alpha, pamax = k1(q, k, src, dst)
        amax = kmax(pamax)
        ex, pden = k3(alpha, dst, amax)
        den = ksum(pden)
        outp = k5(v, ex, src, dst, den)
        kind = min(i, 2)
        cur = _combine(outp, skip, skip if i == 0 else cur, w1p, w2p, bsum,
                       cp, kind)
        fin = cp

    pooled = _k6_pool(cp)(cur, batchp)
    g1w = jnp.pad(g1[0], ((0, 0), (0, cp - c)))
    g = _matmul(pooled, g1w, g1[1], act="relu")
    return _matmul(g, g2[0], g2[1])


def kernel(mol_x, mol_edge_index, mol_batch, clique_x, clique_edge_index,
           clique_batch, data_pre, params):
    p = params
    x = _branch(mol_x, mol_edge_index, mol_batch, p["mol_convs"],
                p["mol_seq_fc1"], p["mol_seq_fc2"], p["mol_bias"],
                p["mol_g1"], p["mol_g2"], 312, 320, 78)
    xq = _branch(clique_x, clique_edge_index, clique_batch, p["cli_convs"],
                 p["cli_seq_fc1"], p["cli_seq_fc2"], p["cli_bias"],
                 p["cli_g1"], p["cli_g2"], 368, 368, 92)
    return _head(x, xq, p)
